# Initial kernel scaffold; baseline (speedup 1.0000x reference)
import jax
import jax.numpy as jnp
from jax.experimental import pallas as pl


def kernel(x, edge_index_r0, edge_index_r1, W_r0, b_r0, W_r1, b_r1):
    raise NotImplementedError("write your pallas kernel here")


# speedup vs baseline: 5.8707x; 5.8707x over previous
import functools

import jax
import jax.numpy as jnp
from jax import lax
from jax.experimental import pallas as pl
from jax.experimental.pallas import tpu as pltpu
from jax.experimental.pallas import tpu_sc as plsc

N = 10000
E = 320000
D = 128
HD = D // 2

NS = 16
EPT = E // NS
K = 80
NCH = EPT // K
HB = 10240
ZB = HB // NS
NPAD = 10240
RPT = NPAD // NS
ZR = 128

_mesh = plsc.VectorSubcoreMesh(core_axis_name="c", subcore_axis_name="s")


def _hist_body(s0_ref, d0_ref, s1_ref, d1_ref, out_ref,
               sidx, didx, ones_v, zb, bins_s, bins_d):
    c = lax.axis_index("c")
    s = lax.axis_index("s")

    zero16 = jnp.zeros((16,), jnp.float32)
    one16 = jnp.ones((16,), jnp.float32)
    for i in range(ZB // 16):
        zb[pl.ds(i * 16, 16)] = zero16
    for i in range(K // 16):
        ones_v[pl.ds(i * 16, 16)] = one16

    pltpu.sync_copy(zb, bins_s.at[pl.ds(s * ZB, ZB)])
    pltpu.sync_copy(zb, bins_d.at[pl.ds(s * ZB, ZB)])

    @pl.when(c == 0)
    def _():
        pltpu.sync_copy(s0_ref.at[s], sidx)
        pltpu.sync_copy(d0_ref.at[s], didx)

    @pl.when(c == 1)
    def _():
        pltpu.sync_copy(s1_ref.at[s], sidx)
        pltpu.sync_copy(d1_ref.at[s], didx)

    plsc.subcore_barrier()

    @pl.loop(0, NCH)
    def _(j):
        pltpu.sync_copy(ones_v, bins_s.at[sidx.at[j]], add=True)
        pltpu.sync_copy(ones_v, bins_d.at[didx.at[j]], add=True)

    plsc.subcore_barrier()

    @pl.when(c == 0)
    def _():
        @pl.when(s == 0)
        def _():
            pltpu.sync_copy(bins_s, out_ref.at[0])

        @pl.when(s == 1)
        def _():
            pltpu.sync_copy(bins_d, out_ref.at[1])

    @pl.when(c == 1)
    def _():
        @pl.when(s == 0)
        def _():
            pltpu.sync_copy(bins_s, out_ref.at[2])

        @pl.when(s == 1)
        def _():
            pltpu.sync_copy(bins_d, out_ref.at[3])


_hist = pl.kernel(
    _hist_body,
    out_type=jax.ShapeDtypeStruct((4, HB), jnp.float32),
    mesh=_mesh,
    scratch_types=[
        pltpu.VMEM((NCH, K), jnp.int32),
        pltpu.VMEM((NCH, K), jnp.int32),
        pltpu.VMEM((K,), jnp.float32),
        pltpu.VMEM((ZB,), jnp.float32),
        pltpu.VMEM_SHARED((HB,), jnp.float32),
        pltpu.VMEM_SHARED((HB,), jnp.float32),
    ],
)


def _agg_body(h0a_ref, h0b_ref, h1a_ref, h1b_ref,
              s0_ref, d0_ref, s1_ref, d1_ref,
              a0a_ref, a0b_ref, a1a_ref, a1b_ref,
              sidx, didx, grow, zrow, aggsp):
    c = lax.axis_index("c")
    s = lax.axis_index("s")

    zero16 = jnp.zeros((16,), jnp.float32)

    @pl.loop(0, ZR)
    def _(r):
        for i in range(HD // 16):
            zrow[r, pl.ds(i * 16, 16)] = zero16

    @pl.when(c == 0)
    def _():
        pltpu.sync_copy(s0_ref.at[s], sidx)
        pltpu.sync_copy(d0_ref.at[s], didx)

    @pl.when(c == 1)
    def _():
        pltpu.sync_copy(s1_ref.at[s], sidx)
        pltpu.sync_copy(d1_ref.at[s], didx)

    h0_refs = (h0a_ref, h0b_ref)
    h1_refs = (h1a_ref, h1b_ref)
    a0_refs = (a0a_ref, a0b_ref)
    a1_refs = (a1a_ref, a1b_ref)

    for p in range(2):
        for i in range(RPT // ZR):
            pltpu.sync_copy(zrow, aggsp.at[pl.ds(s * RPT + i * ZR, ZR)])

        plsc.subcore_barrier()

        @pl.loop(0, NCH)
        def _(j):
            @pl.when(c == 0)
            def _():
                pltpu.sync_copy(h0_refs[p].at[sidx.at[j]], grow)

            @pl.when(c == 1)
            def _():
                pltpu.sync_copy(h1_refs[p].at[sidx.at[j]], grow)

            pltpu.sync_copy(grow, aggsp.at[didx.at[j]], add=True)

        plsc.subcore_barrier()

        @pl.when(c == 0)
        def _():
            pltpu.sync_copy(aggsp.at[pl.ds(s * RPT, RPT)],
                            a0_refs[p].at[pl.ds(s * RPT, RPT)])

        @pl.when(c == 1)
        def _():
            pltpu.sync_copy(aggsp.at[pl.ds(s * RPT, RPT)],
                            a1_refs[p].at[pl.ds(s * RPT, RPT)])


_agg = pl.kernel(
    _agg_body,
    out_type=[jax.ShapeDtypeStruct((NPAD, HD), jnp.float32)] * 4,
    mesh=_mesh,
    compiler_params=pltpu.CompilerParams(use_tc_tiling_on_sc=False),
    scratch_types=[
        pltpu.VMEM((NCH, K), jnp.int32),
        pltpu.VMEM((NCH, K), jnp.int32),
        pltpu.VMEM((K, HD), jnp.float32),
        pltpu.VMEM((ZR, HD), jnp.float32),
        pltpu.VMEM_SHARED((NPAD, HD), jnp.float32),
    ],
)


_RB = 1000


def _scale_body(x_ref, dt_ref, h0a_ref, h0b_ref, h1a_ref, h1b_ref):
    xb = x_ref[...]
    s0 = lax.rsqrt(jnp.maximum(dt_ref[:, 0:1], 1.0))
    s1 = lax.rsqrt(jnp.maximum(dt_ref[:, 2:3], 1.0))
    h0 = xb * s0
    h1 = xb * s1
    h0a_ref[...] = h0[:, :HD]
    h0b_ref[...] = h0[:, HD:]
    h1a_ref[...] = h1[:, :HD]
    h1b_ref[...] = h1[:, HD:]


def _scale(x, degs_t):
    return pl.pallas_call(
        _scale_body,
        grid=(N // _RB,),
        in_specs=[pl.BlockSpec((_RB, D), lambda j: (j, 0)),
                  pl.BlockSpec((_RB, 4), lambda j: (j, 0))],
        out_specs=[pl.BlockSpec((_RB, HD), lambda j: (j, 0))] * 4,
        out_shape=[jax.ShapeDtypeStruct((N, HD), jnp.float32)] * 4,
    )(x, degs_t)


def _finish_body(a0a_ref, a0b_ref, a1a_ref, a1b_ref, dt_ref,
                 w0_ref, w1_ref, b0_ref, b1_ref, out_ref):
    si0 = lax.rsqrt(jnp.maximum(dt_ref[:, 1:2], 1.0))
    si1 = lax.rsqrt(jnp.maximum(dt_ref[:, 3:4], 1.0))
    a0 = jnp.concatenate([a0a_ref[...], a0b_ref[...]], axis=1)
    a1 = jnp.concatenate([a1a_ref[...], a1b_ref[...]], axis=1)
    h = jnp.dot(a0 * si0, w0_ref[...],
                preferred_element_type=jnp.float32) + b0_ref[...]
    h = h + jnp.dot(a1 * si1, w1_ref[...],
                    preferred_element_type=jnp.float32) + b1_ref[...]
    nrm = jnp.maximum(jnp.sqrt(jnp.sum(h * h, axis=-1, keepdims=True)),
                      1e-12)
    out_ref[...] = h / nrm


def _finish(a0a, a0b, a1a, a1b, degs_t, W0, W1, b0, b1):
    return pl.pallas_call(
        _finish_body,
        grid=(N // _RB,),
        in_specs=[pl.BlockSpec((_RB, HD), lambda j: (j, 0)),
                  pl.BlockSpec((_RB, HD), lambda j: (j, 0)),
                  pl.BlockSpec((_RB, HD), lambda j: (j, 0)),
                  pl.BlockSpec((_RB, HD), lambda j: (j, 0)),
                  pl.BlockSpec((_RB, 4), lambda j: (j, 0)),
                  pl.BlockSpec((D, D), lambda j: (0, 0)),
                  pl.BlockSpec((D, D), lambda j: (0, 0)),
                  pl.BlockSpec((1, D), lambda j: (0, 0)),
                  pl.BlockSpec((1, D), lambda j: (0, 0))],
        out_specs=pl.BlockSpec((_RB, D), lambda j: (j, 0)),
        out_shape=jax.ShapeDtypeStruct((N, D), jnp.float32),
    )(a0a, a0b, a1a, a1b, degs_t, W0, W1, b0, b1)


def kernel(x, edge_index_r0, edge_index_r1, W_r0, b_r0, W_r1, b_r1):
    s0r = edge_index_r0[0].reshape(NS, NCH, K)
    d0r = edge_index_r0[1].reshape(NS, NCH, K)
    s1r = edge_index_r1[0].reshape(NS, NCH, K)
    d1r = edge_index_r1[1].reshape(NS, NCH, K)

    degs = _hist(s0r, d0r, s1r, d1r)
    degs_t = degs[:, :N].T

    h0a, h0b, h1a, h1b = _scale(x, degs_t)
    a0a, a0b, a1a, a1b = _agg(h0a, h0b, h1a, h1b, s0r, d0r, s1r, d1r)
    return _finish(a0a, a0b, a1a, a1b, degs_t, W_r0, W_r1,
                   b_r0.reshape(1, D), b_r1.reshape(1, D))


# speedup vs baseline: 12.4861x; 2.1268x over previous
import functools

import jax
import jax.numpy as jnp
from jax import lax
from jax.experimental import pallas as pl
from jax.experimental.pallas import tpu as pltpu
from jax.experimental.pallas import tpu_sc as plsc

N = 10000
E = 320000
D = 128
HD = D // 2

NS = 16
EPT = E // NS
K = 80
NCH = EPT // K
HB = 10240
ZB = HB // NS
NPAD = 10240
RPT = NPAD // NS
ZR = 128

_mesh = plsc.VectorSubcoreMesh(core_axis_name="c", subcore_axis_name="s")


def _hist_body(s0_ref, d0_ref, s1_ref, d1_ref, out_ref,
               sidx, didx, ones_v, zb, bins_s, bins_d):
    c = lax.axis_index("c")
    s = lax.axis_index("s")

    zero16 = jnp.zeros((16,), jnp.float32)
    one16 = jnp.ones((16,), jnp.float32)
    for i in range(ZB // 16):
        zb[pl.ds(i * 16, 16)] = zero16
    for i in range(K // 16):
        ones_v[pl.ds(i * 16, 16)] = one16

    pltpu.sync_copy(zb, bins_s.at[pl.ds(s * ZB, ZB)])
    pltpu.sync_copy(zb, bins_d.at[pl.ds(s * ZB, ZB)])

    @pl.when(c == 0)
    def _():
        pltpu.sync_copy(s0_ref.at[s], sidx)
        pltpu.sync_copy(d0_ref.at[s], didx)

    @pl.when(c == 1)
    def _():
        pltpu.sync_copy(s1_ref.at[s], sidx)
        pltpu.sync_copy(d1_ref.at[s], didx)

    plsc.subcore_barrier()

    @pl.loop(0, NCH)
    def _(j):
        pltpu.sync_copy(ones_v, bins_s.at[sidx.at[j]], add=True)
        pltpu.sync_copy(ones_v, bins_d.at[didx.at[j]], add=True)

    plsc.subcore_barrier()

    @pl.when(c == 0)
    def _():
        @pl.when(s == 0)
        def _():
            pltpu.sync_copy(bins_s, out_ref.at[0])

        @pl.when(s == 1)
        def _():
            pltpu.sync_copy(bins_d, out_ref.at[1])

    @pl.when(c == 1)
    def _():
        @pl.when(s == 0)
        def _():
            pltpu.sync_copy(bins_s, out_ref.at[2])

        @pl.when(s == 1)
        def _():
            pltpu.sync_copy(bins_d, out_ref.at[3])


_hist = pl.kernel(
    _hist_body,
    out_type=jax.ShapeDtypeStruct((4, HB), jnp.float32),
    mesh=_mesh,
    scratch_types=[
        pltpu.VMEM((NCH, K), jnp.int32),
        pltpu.VMEM((NCH, K), jnp.int32),
        pltpu.VMEM((K,), jnp.float32),
        pltpu.VMEM((ZB,), jnp.float32),
        pltpu.VMEM_SHARED((HB,), jnp.float32),
        pltpu.VMEM_SHARED((HB,), jnp.float32),
    ],
)


NBUF = 5


def _agg_body(h0a_ref, h0b_ref, h1a_ref, h1b_ref,
              s0_ref, d0_ref, s1_ref, d1_ref,
              a0a_ref, a0b_ref, a1a_ref, a1b_ref,
              sidx, didx, g0, g1, g2, g3, g4, zrow, aggsp,
              e0, e1, e2, e3, e4):
    c = lax.axis_index("c")
    s = lax.axis_index("s")
    grows = (g0, g1, g2, g3, g4)
    sems = (e0, e1, e2, e3, e4)

    zero16 = jnp.zeros((16,), jnp.float32)

    @pl.loop(0, ZR)
    def _(r):
        for i in range(HD // 16):
            zrow[r, pl.ds(i * 16, 16)] = zero16

    @pl.when(c == 0)
    def _():
        pltpu.sync_copy(s0_ref.at[s], sidx)
        pltpu.sync_copy(d0_ref.at[s], didx)

    @pl.when(c == 1)
    def _():
        pltpu.sync_copy(s1_ref.at[s], sidx)
        pltpu.sync_copy(d1_ref.at[s], didx)

    h0_refs = (h0a_ref, h0b_ref)
    h1_refs = (h1a_ref, h1b_ref)
    a0_refs = (a0a_ref, a0b_ref)
    a1_refs = (a1a_ref, a1b_ref)

    for p in range(2):
        def _start(b, j):
            @pl.when(c == 0)
            def _():
                pltpu.async_copy(h0_refs[p].at[sidx.at[j]], grows[b],
                                 sems[b])

            @pl.when(c == 1)
            def _():
                pltpu.async_copy(h1_refs[p].at[sidx.at[j]], grows[b],
                                 sems[b])

        def _wait(b, j):
            pltpu.make_async_copy(h0_refs[p].at[sidx.at[j]], grows[b],
                                  sems[b]).wait()

        for b in range(NBUF):
            _start(b, b)

        for i in range(RPT // ZR):
            pltpu.sync_copy(zrow, aggsp.at[pl.ds(s * RPT + i * ZR, ZR)])

        plsc.subcore_barrier()

        @pl.loop(0, NCH, step=NBUF)
        def _(j):
            for b in range(NBUF):
                _wait(b, j + b)
                pltpu.sync_copy(grows[b], aggsp.at[didx.at[j + b]],
                                add=True)

                @pl.when(j + b + NBUF < NCH)
                def _():
                    _start(b, j + b + NBUF)

        plsc.subcore_barrier()

        @pl.when(c == 0)
        def _():
            pltpu.sync_copy(aggsp.at[pl.ds(s * RPT, RPT)],
                            a0_refs[p].at[pl.ds(s * RPT, RPT)])

        @pl.when(c == 1)
        def _():
            pltpu.sync_copy(aggsp.at[pl.ds(s * RPT, RPT)],
                            a1_refs[p].at[pl.ds(s * RPT, RPT)])


_agg = pl.kernel(
    _agg_body,
    out_type=[jax.ShapeDtypeStruct((NPAD, HD), jnp.float32)] * 4,
    mesh=_mesh,
    compiler_params=pltpu.CompilerParams(use_tc_tiling_on_sc=False),
    scratch_types=[
        pltpu.VMEM((NCH, K), jnp.int32),
        pltpu.VMEM((NCH, K), jnp.int32),
    ] + [pltpu.VMEM((K, HD), jnp.float32)] * NBUF +
    [
        pltpu.VMEM((ZR, HD), jnp.float32),
        pltpu.VMEM_SHARED((NPAD, HD), jnp.float32),
    ] + [pltpu.SemaphoreType.DMA] * NBUF,
)


_RB = 1000


def _scale_body(x_ref, dt_ref, h0a_ref, h0b_ref, h1a_ref, h1b_ref):
    xb = x_ref[...]
    s0 = lax.rsqrt(jnp.maximum(dt_ref[:, 0:1], 1.0))
    s1 = lax.rsqrt(jnp.maximum(dt_ref[:, 2:3], 1.0))
    h0 = xb * s0
    h1 = xb * s1
    h0a_ref[...] = h0[:, :HD]
    h0b_ref[...] = h0[:, HD:]
    h1a_ref[...] = h1[:, :HD]
    h1b_ref[...] = h1[:, HD:]


def _scale(x, degs_t):
    return pl.pallas_call(
        _scale_body,
        grid=(N // _RB,),
        in_specs=[pl.BlockSpec((_RB, D), lambda j: (j, 0)),
                  pl.BlockSpec((_RB, 4), lambda j: (j, 0))],
        out_specs=[pl.BlockSpec((_RB, HD), lambda j: (j, 0))] * 4,
        out_shape=[jax.ShapeDtypeStruct((N, HD), jnp.float32)] * 4,
    )(x, degs_t)


def _finish_body(a0a_ref, a0b_ref, a1a_ref, a1b_ref, dt_ref,
                 w0_ref, w1_ref, b0_ref, b1_ref, out_ref):
    si0 = lax.rsqrt(jnp.maximum(dt_ref[:, 1:2], 1.0))
    si1 = lax.rsqrt(jnp.maximum(dt_ref[:, 3:4], 1.0))
    a0 = jnp.concatenate([a0a_ref[...], a0b_ref[...]], axis=1)
    a1 = jnp.concatenate([a1a_ref[...], a1b_ref[...]], axis=1)
    h = jnp.dot(a0 * si0, w0_ref[...],
                preferred_element_type=jnp.float32) + b0_ref[...]
    h = h + jnp.dot(a1 * si1, w1_ref[...],
                    preferred_element_type=jnp.float32) + b1_ref[...]
    nrm = jnp.maximum(jnp.sqrt(jnp.sum(h * h, axis=-1, keepdims=True)),
                      1e-12)
    out_ref[...] = h / nrm


def _finish(a0a, a0b, a1a, a1b, degs_t, W0, W1, b0, b1):
    return pl.pallas_call(
        _finish_body,
        grid=(N // _RB,),
        in_specs=[pl.BlockSpec((_RB, HD), lambda j: (j, 0)),
                  pl.BlockSpec((_RB, HD), lambda j: (j, 0)),
                  pl.BlockSpec((_RB, HD), lambda j: (j, 0)),
                  pl.BlockSpec((_RB, HD), lambda j: (j, 0)),
                  pl.BlockSpec((_RB, 4), lambda j: (j, 0)),
                  pl.BlockSpec((D, D), lambda j: (0, 0)),
                  pl.BlockSpec((D, D), lambda j: (0, 0)),
                  pl.BlockSpec((1, D), lambda j: (0, 0)),
                  pl.BlockSpec((1, D), lambda j: (0, 0))],
        out_specs=pl.BlockSpec((_RB, D), lambda j: (j, 0)),
        out_shape=jax.ShapeDtypeStruct((N, D), jnp.float32),
    )(a0a, a0b, a1a, a1b, degs_t, W0, W1, b0, b1)


def kernel(x, edge_index_r0, edge_index_r1, W_r0, b_r0, W_r1, b_r1):
    s0r = edge_index_r0[0].reshape(NS, NCH, K)
    d0r = edge_index_r0[1].reshape(NS, NCH, K)
    s1r = edge_index_r1[0].reshape(NS, NCH, K)
    d1r = edge_index_r1[1].reshape(NS, NCH, K)

    degs = _hist(s0r, d0r, s1r, d1r)
    degs_t = degs[:, :N].T

    h0a, h0b, h1a, h1b = _scale(x, degs_t)
    a0a, a0b, a1a, a1b = _agg(h0a, h0b, h1a, h1b, s0r, d0r, s1r, d1r)
    return _finish(a0a, a0b, a1a, a1b, degs_t, W_r0, W_r1,
                   b_r0.reshape(1, D), b_r1.reshape(1, D))
